# baseline (device time: 31325 ns/iter reference)
import jax
import jax.numpy as jnp
from jax import lax
from jax.experimental import pallas as pl
from jax.experimental.pallas import tpu as pltpu

N_DEV = 8
B = 512
NSI = 8
NSO = 4
LOOKAHEAD = 6


def kernel(x):
    m, n = x.shape
    assert m % B == 0
    C = m // B
    assert C >= 3
    f32 = x.dtype
    bf16 = jnp.bfloat16

    def body(x_hbm, out_hbm, xbuf, obuf, lo_ref, hi_ref,
             in_sems, out_sems, send_sems, recv_sems):
        my = lax.axis_index("i")

        @pl.when(my > 0)
        def _():
            pltpu.make_async_remote_copy(
                src_ref=x_hbm.at[pl.ds(0, 1)],
                dst_ref=hi_ref,
                send_sem=send_sems.at[0],
                recv_sem=recv_sems.at[1],
                device_id=(my - 1,),
                device_id_type=pltpu.DeviceIdType.MESH,
            ).start()

        @pl.when(my < N_DEV - 1)
        def _():
            pltpu.make_async_remote_copy(
                src_ref=x_hbm.at[pl.ds(m - 1, 1)],
                dst_ref=lo_ref,
                send_sem=send_sems.at[1],
                recv_sem=recv_sems.at[0],
                device_id=(my + 1,),
                device_id_type=pltpu.DeviceIdType.MESH,
            ).start()

        def in_copy(c):
            return pltpu.make_async_copy(
                x_hbm.at[pl.ds(c * B, B)],
                xbuf.at[c % NSI],
                in_sems.at[c % NSI],
            )

        def out_copy(c):
            return pltpu.make_async_copy(
                obuf.at[c % NSO],
                out_hbm.at[pl.ds(c * B, B)],
                out_sems.at[c % NSO],
            )

        for c in range(min(LOOKAHEAD, C)):
            in_copy(c).start()

        for i in range(C):
            slot, prev, nxt, oslot = i % NSI, (i - 1) % NSI, (i + 1) % NSI, i % NSO
            if i + LOOKAHEAD < C:
                in_copy(i + LOOKAHEAD).start()
            if i == 0:
                in_copy(0).wait()
            if i + 1 < C:
                in_copy(i + 1).wait()

            if i >= NSO:
                out_copy(i - NSO).wait()

            obuf[oslot, pl.ds(1, B - 2)] = (
                0.25 * xbuf[slot, pl.ds(0, B - 2)]
                + 0.5 * xbuf[slot, pl.ds(1, B - 2)]
                + 0.25 * xbuf[slot, pl.ds(2, B - 2)]
            ).astype(bf16)

            if i == 0:
                @pl.when(my > 0)
                def _():
                    pltpu.make_async_remote_copy(
                        src_ref=x_hbm.at[pl.ds(0, 1)],
                        dst_ref=lo_ref,
                        send_sem=send_sems.at[0],
                        recv_sem=recv_sems.at[0],
                        device_id=(my - 1,),
                        device_id_type=pltpu.DeviceIdType.MESH,
                    ).wait_recv()
                    obuf[oslot, pl.ds(0, 1)] = (
                        0.25 * lo_ref[...]
                        + 0.5 * xbuf[slot, pl.ds(0, 1)]
                        + 0.25 * xbuf[slot, pl.ds(1, 1)]
                    ).astype(bf16)

                @pl.when(my == 0)
                def _():
                    obuf[oslot, pl.ds(0, 1)] = (
                        xbuf[slot, pl.ds(0, 1)].astype(bf16)
                    )
            else:
                obuf[oslot, pl.ds(0, 1)] = (
                    0.25 * xbuf[prev, pl.ds(B - 1, 1)]
                    + 0.5 * xbuf[slot, pl.ds(0, 1)]
                    + 0.25 * xbuf[slot, pl.ds(1, 1)]
                ).astype(bf16)

            if i == C - 1:
                @pl.when(my < N_DEV - 1)
                def _():
                    pltpu.make_async_remote_copy(
                        src_ref=x_hbm.at[pl.ds(0, 1)],
                        dst_ref=hi_ref,
                        send_sem=send_sems.at[0],
                        recv_sem=recv_sems.at[1],
                        device_id=(my + 1,),
                        device_id_type=pltpu.DeviceIdType.MESH,
                    ).wait_recv()
                    obuf[oslot, pl.ds(B - 1, 1)] = (
                        0.25 * xbuf[slot, pl.ds(B - 2, 1)]
                        + 0.5 * xbuf[slot, pl.ds(B - 1, 1)]
                        + 0.25 * hi_ref[...]
                    ).astype(bf16)

                @pl.when(my == N_DEV - 1)
                def _():
                    obuf[oslot, pl.ds(B - 1, 1)] = (
                        xbuf[slot, pl.ds(B - 1, 1)].astype(bf16)
                    )
            else:
                obuf[oslot, pl.ds(B - 1, 1)] = (
                    0.25 * xbuf[slot, pl.ds(B - 2, 1)]
                    + 0.5 * xbuf[slot, pl.ds(B - 1, 1)]
                    + 0.25 * xbuf[nxt, pl.ds(0, 1)]
                ).astype(bf16)

            out_copy(i).start()

        for c in range(max(C - NSO, 0), C):
            out_copy(c).wait()

        @pl.when(my > 0)
        def _():
            pltpu.make_async_remote_copy(
                src_ref=x_hbm.at[pl.ds(0, 1)],
                dst_ref=hi_ref,
                send_sem=send_sems.at[0],
                recv_sem=recv_sems.at[1],
                device_id=(my - 1,),
                device_id_type=pltpu.DeviceIdType.MESH,
            ).wait_send()

        @pl.when(my < N_DEV - 1)
        def _():
            pltpu.make_async_remote_copy(
                src_ref=x_hbm.at[pl.ds(m - 1, 1)],
                dst_ref=lo_ref,
                send_sem=send_sems.at[1],
                recv_sem=recv_sems.at[0],
                device_id=(my + 1,),
                device_id_type=pltpu.DeviceIdType.MESH,
            ).wait_send()

    return pl.pallas_call(
        body,
        out_shape=jax.ShapeDtypeStruct((m, n), bf16),
        in_specs=[pl.BlockSpec(memory_space=pl.ANY)],
        out_specs=pl.BlockSpec(memory_space=pl.ANY),
        scratch_shapes=[
            pltpu.VMEM((NSI, B, n), f32),
            pltpu.VMEM((NSO, B, n), bf16),
            pltpu.VMEM((1, n), f32),
            pltpu.VMEM((1, n), f32),
            pltpu.SemaphoreType.DMA((NSI,)),
            pltpu.SemaphoreType.DMA((NSO,)),
            pltpu.SemaphoreType.DMA((2,)),
            pltpu.SemaphoreType.DMA((2,)),
        ],
    )(x)
